# Initial kernel scaffold; baseline (speedup 1.0000x reference)
import jax
import jax.numpy as jnp
from jax.experimental import pallas as pl


def kernel(edge_features, edge_index, node_degrees, batch, heads, tails, g_all_init, Wa1, ba1, Wa2, ba2, Wu1, bu1, Wu2, bu2, Wg1, bg1, Wg2, bg2, We, be):
    raise NotImplementedError("write your pallas kernel here")


# speedup vs baseline: 41.9540x; 41.9540x over previous
import functools

import jax
import jax.numpy as jnp
from jax.experimental import pallas as pl
from jax.experimental.pallas import tpu as pltpu

H = 128
BE = 2000
BN = 2000


def _attn_body(be_ref, Wa1_ref, ba1_ref, Wa2_ref, ba2_ref, V_ref, ex_ref):
    x = be_ref[...]
    h = jnp.maximum(jnp.dot(x, Wa1_ref[...]) + ba1_ref[...], 0.0)
    a = jax.nn.sigmoid(jnp.dot(h, Wa2_ref[...]) + ba2_ref[...])
    ex = jnp.exp(a)
    V_ref[...] = x * ex
    ex_ref[...] = jnp.broadcast_to(ex, ex_ref.shape)


def _attn(b_e, Wa1, ba1, Wa2, ba2):
    e = b_e.shape[0]
    grid = e // BE
    return pl.pallas_call(
        _attn_body,
        grid=(grid,),
        in_specs=[
            pl.BlockSpec((BE, H), lambda i: (i, 0)),
            pl.BlockSpec((H, H), lambda i: (0, 0)),
            pl.BlockSpec((1, H), lambda i: (0, 0)),
            pl.BlockSpec((H, 1), lambda i: (0, 0)),
            pl.BlockSpec((1, 1), lambda i: (0, 0)),
        ],
        out_specs=[
            pl.BlockSpec((BE, H), lambda i: (i, 0)),
            pl.BlockSpec((BE, 8), lambda i: (i, 0)),
        ],
        out_shape=[
            jax.ShapeDtypeStruct((e, H), jnp.float32),
            jax.ShapeDtypeStruct((e, 8), jnp.float32),
        ],
    )(b_e, Wa1, ba1.reshape(1, H), Wa2, ba2.reshape(1, 1))


def _node_body(S_ref, s_ref, deg_ref, heads_ref, tails_ref,
               Wsrc_ref, whsrc_ref, wtsrc_ref, Wdst_ref, whdst_ref, wtdst_ref,
               Q_ref, R_ref, bv_ref):
    i = pl.program_id(0)
    S = S_ref[...]
    s1 = s_ref[..., 0:1]
    deg = deg_ref[...]
    bv = jnp.where(s1 > 0.0, S / jnp.where(s1 > 0.0, s1, 1.0), 0.0)
    bv = bv / (1.0 + deg)
    ids = i * BN + jax.lax.broadcasted_iota(jnp.int32, (BN, 1), 0)
    ih = jnp.minimum(jnp.sum((ids == heads_ref[...]).astype(jnp.float32),
                             axis=1, keepdims=True), 1.0)
    it = jnp.minimum(jnp.sum((ids == tails_ref[...]).astype(jnp.float32),
                             axis=1, keepdims=True), 1.0)
    Q_ref[...] = jnp.dot(bv, Wsrc_ref[...]) + ih * whsrc_ref[...] + it * wtsrc_ref[...]
    R_ref[...] = jnp.dot(bv, Wdst_ref[...]) + ih * whdst_ref[...] + it * wtdst_ref[...]
    bv_ref[...] = bv


def _node_tables(S, s, deg, heads, tails, Wsrc, whsrc, wtsrc, Wdst, whdst, wtdst):
    n = S.shape[0]
    nb = heads.shape[0]
    grid = n // BN
    return pl.pallas_call(
        _node_body,
        grid=(grid,),
        in_specs=[
            pl.BlockSpec((BN, H), lambda i: (i, 0)),
            pl.BlockSpec((BN, 8), lambda i: (i, 0)),
            pl.BlockSpec((BN, 1), lambda i: (i, 0)),
            pl.BlockSpec((1, nb), lambda i: (0, 0)),
            pl.BlockSpec((1, nb), lambda i: (0, 0)),
            pl.BlockSpec((H, H), lambda i: (0, 0)),
            pl.BlockSpec((1, H), lambda i: (0, 0)),
            pl.BlockSpec((1, H), lambda i: (0, 0)),
            pl.BlockSpec((H, H), lambda i: (0, 0)),
            pl.BlockSpec((1, H), lambda i: (0, 0)),
            pl.BlockSpec((1, H), lambda i: (0, 0)),
        ],
        out_specs=[
            pl.BlockSpec((BN, H), lambda i: (i, 0)),
            pl.BlockSpec((BN, H), lambda i: (i, 0)),
            pl.BlockSpec((BN, H), lambda i: (i, 0)),
        ],
        out_shape=[
            jax.ShapeDtypeStruct((n, H), jnp.float32),
            jax.ShapeDtypeStruct((n, H), jnp.float32),
            jax.ShapeDtypeStruct((n, H), jnp.float32),
        ],
    )(S, s, deg.reshape(n, 1), heads.reshape(1, nb), tails.reshape(1, nb),
      Wsrc, whsrc, wtsrc, Wdst, whdst, wtdst)


def _update_body(gs_ref, gd_ref, be_ref, Wue_ref, bu1_ref, Wu2_ref, bu2_ref, out_ref):
    z = gs_ref[...] + gd_ref[...] + jnp.dot(be_ref[...], Wue_ref[...]) + bu1_ref[...]
    out_ref[...] = jnp.dot(jnp.maximum(z, 0.0), Wu2_ref[...]) + bu2_ref[...]


def _update(gs, gd, b_e, Wue, bu1, Wu2, bu2):
    e = gs.shape[0]
    grid = e // BE
    return pl.pallas_call(
        _update_body,
        grid=(grid,),
        in_specs=[
            pl.BlockSpec((BE, H), lambda i: (i, 0)),
            pl.BlockSpec((BE, H), lambda i: (i, 0)),
            pl.BlockSpec((BE, H), lambda i: (i, 0)),
            pl.BlockSpec((H, H), lambda i: (0, 0)),
            pl.BlockSpec((1, H), lambda i: (0, 0)),
            pl.BlockSpec((H, H), lambda i: (0, 0)),
            pl.BlockSpec((1, H), lambda i: (0, 0)),
        ],
        out_specs=pl.BlockSpec((BE, H), lambda i: (i, 0)),
        out_shape=jax.ShapeDtypeStruct((e, H), jnp.float32),
    )(gs, gd, b_e, Wue, bu1.reshape(1, H), Wu2, bu2.reshape(1, H))


def _update_we_body(gs_ref, gd_ref, be_ref, Wue_ref, bu1_ref, Wu2_ref, bu2_ref,
                    We_ref, be2_ref, we_ref):
    z = gs_ref[...] + gd_ref[...] + jnp.dot(be_ref[...], Wue_ref[...]) + bu1_ref[...]
    bnew = jnp.dot(jnp.maximum(z, 0.0), Wu2_ref[...]) + bu2_ref[...]
    we_ref[...] = jax.nn.sigmoid(jnp.dot(bnew, We_ref[...]) + be2_ref[...])


def _update_we(gs, gd, b_e, Wue, bu1, Wu2, bu2, We, be):
    e = gs.shape[0]
    grid = e // BE
    return pl.pallas_call(
        _update_we_body,
        grid=(grid,),
        in_specs=[
            pl.BlockSpec((BE, H), lambda i: (i, 0)),
            pl.BlockSpec((BE, H), lambda i: (i, 0)),
            pl.BlockSpec((BE, H), lambda i: (i, 0)),
            pl.BlockSpec((H, H), lambda i: (0, 0)),
            pl.BlockSpec((1, H), lambda i: (0, 0)),
            pl.BlockSpec((H, H), lambda i: (0, 0)),
            pl.BlockSpec((1, H), lambda i: (0, 0)),
            pl.BlockSpec((H, 1), lambda i: (0, 0)),
            pl.BlockSpec((1, 1), lambda i: (0, 0)),
        ],
        out_specs=pl.BlockSpec((BE, 1), lambda i: (i, 0)),
        out_shape=jax.ShapeDtypeStruct((e, 1), jnp.float32),
    )(gs, gd, b_e, Wue, bu1.reshape(1, H), Wu2, bu2.reshape(1, H),
      We, be.reshape(1, 1))


def _final_body(bv_ref, batch_ref, heads_ref, tails_ref,
                Wg1_ref, bg1_ref, Wg2_ref, bg2_ref, gall_ref, gm_ref):
    bv = bv_ref[...]
    n = bv.shape[0]
    nb = heads_ref.shape[0]
    batch = batch_ref[...]

    def body(b):
        mask = batch == b
        m = jnp.max(jnp.where(mask, bv, -jnp.inf), axis=0)
        gm_ref[pl.ds(b, 1), :] = m[None, :]
        return b + jnp.int32(1)

    jax.lax.while_loop(lambda b: b < nb, body, jnp.int32(0))
    gmax = gm_ref[...]
    gmax = jnp.where(jnp.isfinite(gmax), gmax, 0.0)

    ids = jax.lax.broadcasted_iota(jnp.int32, (nb, n), 1)
    oh_h = (ids == heads_ref[...]).astype(jnp.float32)
    oh_t = (ids == tails_ref[...]).astype(jnp.float32)
    bh = jnp.dot(oh_h, bv)
    bt = jnp.dot(oh_t, bv)
    g_G = jnp.concatenate([gmax, bh, bt], axis=1)
    hmid = jnp.maximum(jnp.dot(g_G, Wg1_ref[...]) + bg1_ref[...], 0.0)
    logits = jnp.dot(hmid, Wg2_ref[...]) + bg2_ref[...]
    w = jax.nn.softmax(logits, axis=0)
    gall_ref[...] = jnp.sum(w * g_G, axis=0, keepdims=True)


def _final(bv, batch, heads, tails, Wg1, bg1, Wg2, bg2):
    n = bv.shape[0]
    nb = heads.shape[0]
    return pl.pallas_call(
        _final_body,
        in_specs=[
            pl.BlockSpec((n, H), lambda: (0, 0)),
            pl.BlockSpec((n, 1), lambda: (0, 0)),
            pl.BlockSpec((nb, 1), lambda: (0, 0)),
            pl.BlockSpec((nb, 1), lambda: (0, 0)),
            pl.BlockSpec((3 * H, H), lambda: (0, 0)),
            pl.BlockSpec((1, H), lambda: (0, 0)),
            pl.BlockSpec((H, 1), lambda: (0, 0)),
            pl.BlockSpec((1, 1), lambda: (0, 0)),
        ],
        out_specs=pl.BlockSpec((1, 3 * H), lambda: (0, 0)),
        out_shape=jax.ShapeDtypeStruct((1, 3 * H), jnp.float32),
        scratch_shapes=[pltpu.VMEM((nb, H), jnp.float32)],
    )(bv, batch.reshape(n, 1), heads.reshape(nb, 1), tails.reshape(nb, 1),
      Wg1, bg1.reshape(1, H), Wg2, bg2.reshape(1, 1))


def kernel(edge_features, edge_index, node_degrees, batch, heads, tails,
           g_all_init, Wa1, ba1, Wa2, ba2, Wu1, bu1, Wu2, bu2,
           Wg1, bg1, Wg2, bg2, We, be):
    e = edge_features.shape[0]
    n = node_degrees.shape[0]
    f32 = jnp.float32
    with jax.enable_x64(False):
        (edge_features, node_degrees, g_all_init, Wa1, ba1, Wa2, ba2, Wu1,
         bu1, Wu2, bu2, Wg1, bg1, Wg2, bg2, We, be) = jax.tree.map(
            lambda x: x.astype(f32),
            (edge_features, node_degrees, g_all_init, Wa1, ba1, Wa2, ba2, Wu1,
             bu1, Wu2, bu2, Wg1, bg1, Wg2, bg2, We, be))
        src = edge_index[0].astype(jnp.int32)
        dst = edge_index[1].astype(jnp.int32)
        heads32 = heads.astype(jnp.int32)
        tails32 = tails.astype(jnp.int32)
        batch32 = batch.astype(jnp.int32)

        Wsrc, whsrc, wtsrc = Wu1[0:H], Wu1[H:H + 1], Wu1[H + 1:H + 2]
        Wdst, whdst, wtdst = (Wu1[H + 2:2 * H + 2], Wu1[2 * H + 2:2 * H + 3],
                              Wu1[2 * H + 3:2 * H + 4])
        Wue = Wu1[2 * H + 4:]

        b_e = jnp.concatenate(
            [edge_features,
             jnp.broadcast_to(g_all_init, (e, g_all_init.shape[-1]))], axis=1)

        bv = None
        for it in range(2):
            V, exv = _attn(b_e, Wa1, ba1, Wa2, ba2)
            S = jax.ops.segment_sum(V, dst, num_segments=n)
            s = jax.ops.segment_sum(exv, dst, num_segments=n)
            Q, R, bv = _node_tables(S, s, node_degrees, heads32, tails32,
                                    Wsrc, whsrc, wtsrc, Wdst, whdst, wtdst)
            gs = jnp.take(Q, src, axis=0)
            gd = jnp.take(R, dst, axis=0)
            if it == 0:
                b_e = _update(gs, gd, b_e, Wue, bu1, Wu2, bu2)
            else:
                w_e = _update_we(gs, gd, b_e, Wue, bu1, Wu2, bu2, We, be)

        g_all = _final(bv, batch32, heads32, tails32, Wg1, bg1, Wg2, bg2)
    return (w_e.astype(jnp.float64), g_all.astype(jnp.float64))


# speedup vs baseline: 56.8841x; 1.3559x over previous
import functools

import jax
import jax.numpy as jnp
from jax import lax
from jax.experimental import pallas as pl
from jax.experimental.pallas import tpu as pltpu
from jax.experimental.pallas import tpu_sc as plsc

H = 128
BE = 2000
BN = 2000


_NC = 2
_NS = 16
_NW = _NC * _NS
_CK = 128


def _sc_gather_add(Q, R, src, dst):
    e = src.shape[0]
    per_w = e // _NW
    nchunk = per_w // _CK
    tail = per_w - nchunk * _CK
    mesh = plsc.VectorSubcoreMesh(core_axis_name="c", subcore_axis_name="s")

    @functools.partial(
        pl.kernel, mesh=mesh,
        out_type=jax.ShapeDtypeStruct((e, H), jnp.float32),
        scratch_types=[
            pltpu.VMEM((_CK,), jnp.int32),
            pltpu.VMEM((_CK,), jnp.int32),
            pltpu.VMEM((_CK, H), jnp.float32),
            pltpu.VMEM((_CK, H), jnp.float32),
            pltpu.SemaphoreType.DMA,
            pltpu.SemaphoreType.DMA,
        ],
    )
    def k(q_hbm, r_hbm, src_hbm, dst_hbm, out_hbm,
          idx_s, idx_d, rows_q, rows_d, sem1, sem2):
        wid = lax.axis_index("s") * _NC + lax.axis_index("c")
        base = wid * per_w

        def do_chunk(cb, ck):
            isl = idx_s.at[pl.ds(0, ck)]
            idl = idx_d.at[pl.ds(0, ck)]
            rq = rows_q.at[pl.ds(0, ck)]
            rd = rows_d.at[pl.ds(0, ck)]
            pltpu.sync_copy(src_hbm.at[pl.ds(cb, ck)], isl)
            pltpu.sync_copy(dst_hbm.at[pl.ds(cb, ck)], idl)
            cp1 = pltpu.async_copy(q_hbm.at[isl], rq, sem1)
            cp2 = pltpu.async_copy(r_hbm.at[idl], rd, sem2)
            cp1.wait()
            cp2.wait()

            def row_body(rr, c):
                for j in range(H // 16):
                    sl = pl.ds(j * 16, 16)
                    rows_q[rr, sl] = rows_q[rr, sl] + rows_d[rr, sl]
                return c

            lax.fori_loop(0, ck, row_body, 0)
            pltpu.sync_copy(rq, out_hbm.at[pl.ds(cb, ck)])

        def chunk_body(kk, c):
            do_chunk(base + kk * _CK, _CK)
            return c

        lax.fori_loop(0, nchunk, chunk_body, 0)
        if tail:
            do_chunk(base + nchunk * _CK, tail)

    return k(Q, R, src, dst)


def _attn_body(be_ref, Wa1_ref, ba1_ref, Wa2_ref, ba2_ref, V_ref, ex_ref):
    x = be_ref[...]
    h = jnp.maximum(jnp.dot(x, Wa1_ref[...]) + ba1_ref[...], 0.0)
    a = jax.nn.sigmoid(jnp.dot(h, Wa2_ref[...]) + ba2_ref[...])
    ex = jnp.exp(a)
    V_ref[...] = x * ex
    ex_ref[...] = jnp.broadcast_to(ex, ex_ref.shape)


def _attn(b_e, Wa1, ba1, Wa2, ba2):
    e = b_e.shape[0]
    grid = e // BE
    return pl.pallas_call(
        _attn_body,
        grid=(grid,),
        in_specs=[
            pl.BlockSpec((BE, H), lambda i: (i, 0)),
            pl.BlockSpec((H, H), lambda i: (0, 0)),
            pl.BlockSpec((1, H), lambda i: (0, 0)),
            pl.BlockSpec((H, 1), lambda i: (0, 0)),
            pl.BlockSpec((1, 1), lambda i: (0, 0)),
        ],
        out_specs=[
            pl.BlockSpec((BE, H), lambda i: (i, 0)),
            pl.BlockSpec((BE, 8), lambda i: (i, 0)),
        ],
        out_shape=[
            jax.ShapeDtypeStruct((e, H), jnp.float32),
            jax.ShapeDtypeStruct((e, 8), jnp.float32),
        ],
    )(b_e, Wa1, ba1.reshape(1, H), Wa2, ba2.reshape(1, 1))


def _node_body(S_ref, s_ref, deg_ref, heads_ref, tails_ref,
               Wsrc_ref, whsrc_ref, wtsrc_ref, Wdst_ref, whdst_ref, wtdst_ref,
               Q_ref, R_ref, bv_ref):
    i = pl.program_id(0)
    S = S_ref[...]
    s1 = s_ref[..., 0:1]
    deg = deg_ref[...]
    bv = jnp.where(s1 > 0.0, S / jnp.where(s1 > 0.0, s1, 1.0), 0.0)
    bv = bv / (1.0 + deg)
    ids = i * BN + jax.lax.broadcasted_iota(jnp.int32, (BN, 1), 0)
    ih = jnp.minimum(jnp.sum((ids == heads_ref[...]).astype(jnp.float32),
                             axis=1, keepdims=True), 1.0)
    it = jnp.minimum(jnp.sum((ids == tails_ref[...]).astype(jnp.float32),
                             axis=1, keepdims=True), 1.0)
    Q_ref[...] = jnp.dot(bv, Wsrc_ref[...]) + ih * whsrc_ref[...] + it * wtsrc_ref[...]
    R_ref[...] = jnp.dot(bv, Wdst_ref[...]) + ih * whdst_ref[...] + it * wtdst_ref[...]
    bv_ref[...] = bv


def _node_tables(S, s, deg, heads, tails, Wsrc, whsrc, wtsrc, Wdst, whdst, wtdst):
    n = S.shape[0]
    nb = heads.shape[0]
    grid = n // BN
    return pl.pallas_call(
        _node_body,
        grid=(grid,),
        in_specs=[
            pl.BlockSpec((BN, H), lambda i: (i, 0)),
            pl.BlockSpec((BN, 8), lambda i: (i, 0)),
            pl.BlockSpec((BN, 1), lambda i: (i, 0)),
            pl.BlockSpec((1, nb), lambda i: (0, 0)),
            pl.BlockSpec((1, nb), lambda i: (0, 0)),
            pl.BlockSpec((H, H), lambda i: (0, 0)),
            pl.BlockSpec((1, H), lambda i: (0, 0)),
            pl.BlockSpec((1, H), lambda i: (0, 0)),
            pl.BlockSpec((H, H), lambda i: (0, 0)),
            pl.BlockSpec((1, H), lambda i: (0, 0)),
            pl.BlockSpec((1, H), lambda i: (0, 0)),
        ],
        out_specs=[
            pl.BlockSpec((BN, H), lambda i: (i, 0)),
            pl.BlockSpec((BN, H), lambda i: (i, 0)),
            pl.BlockSpec((BN, H), lambda i: (i, 0)),
        ],
        out_shape=[
            jax.ShapeDtypeStruct((n, H), jnp.float32),
            jax.ShapeDtypeStruct((n, H), jnp.float32),
            jax.ShapeDtypeStruct((n, H), jnp.float32),
        ],
    )(S, s, deg.reshape(n, 1), heads.reshape(1, nb), tails.reshape(1, nb),
      Wsrc, whsrc, wtsrc, Wdst, whdst, wtdst)


def _update_body(g_ref, be_ref, Wue_ref, bu1_ref, Wu2_ref, bu2_ref, out_ref):
    z = g_ref[...] + jnp.dot(be_ref[...], Wue_ref[...]) + bu1_ref[...]
    out_ref[...] = jnp.dot(jnp.maximum(z, 0.0), Wu2_ref[...]) + bu2_ref[...]


def _update(g, b_e, Wue, bu1, Wu2, bu2):
    e = g.shape[0]
    grid = e // BE
    return pl.pallas_call(
        _update_body,
        grid=(grid,),
        in_specs=[
            pl.BlockSpec((BE, H), lambda i: (i, 0)),
            pl.BlockSpec((BE, H), lambda i: (i, 0)),
            pl.BlockSpec((H, H), lambda i: (0, 0)),
            pl.BlockSpec((1, H), lambda i: (0, 0)),
            pl.BlockSpec((H, H), lambda i: (0, 0)),
            pl.BlockSpec((1, H), lambda i: (0, 0)),
        ],
        out_specs=pl.BlockSpec((BE, H), lambda i: (i, 0)),
        out_shape=jax.ShapeDtypeStruct((e, H), jnp.float32),
    )(g, b_e, Wue, bu1.reshape(1, H), Wu2, bu2.reshape(1, H))


def _update_we_body(g_ref, be_ref, Wue_ref, bu1_ref, Wu2_ref, bu2_ref,
                    We_ref, be2_ref, we_ref):
    z = g_ref[...] + jnp.dot(be_ref[...], Wue_ref[...]) + bu1_ref[...]
    bnew = jnp.dot(jnp.maximum(z, 0.0), Wu2_ref[...]) + bu2_ref[...]
    we_ref[...] = jax.nn.sigmoid(jnp.dot(bnew, We_ref[...]) + be2_ref[...])


def _update_we(g, b_e, Wue, bu1, Wu2, bu2, We, be):
    e = g.shape[0]
    grid = e // BE
    return pl.pallas_call(
        _update_we_body,
        grid=(grid,),
        in_specs=[
            pl.BlockSpec((BE, H), lambda i: (i, 0)),
            pl.BlockSpec((BE, H), lambda i: (i, 0)),
            pl.BlockSpec((H, H), lambda i: (0, 0)),
            pl.BlockSpec((1, H), lambda i: (0, 0)),
            pl.BlockSpec((H, H), lambda i: (0, 0)),
            pl.BlockSpec((1, H), lambda i: (0, 0)),
            pl.BlockSpec((H, 1), lambda i: (0, 0)),
            pl.BlockSpec((1, 1), lambda i: (0, 0)),
        ],
        out_specs=pl.BlockSpec((BE, 1), lambda i: (i, 0)),
        out_shape=jax.ShapeDtypeStruct((e, 1), jnp.float32),
    )(g, b_e, Wue, bu1.reshape(1, H), Wu2, bu2.reshape(1, H),
      We, be.reshape(1, 1))


def _final_body(bv_ref, batch_ref, heads_ref, tails_ref,
                Wg1_ref, bg1_ref, Wg2_ref, bg2_ref, gall_ref, gm_ref):
    bv = bv_ref[...]
    n = bv.shape[0]
    nb = heads_ref.shape[0]
    batch = batch_ref[...]

    def body(b):
        mask = batch == b
        m = jnp.max(jnp.where(mask, bv, -jnp.inf), axis=0)
        gm_ref[pl.ds(b, 1), :] = m[None, :]
        return b + jnp.int32(1)

    jax.lax.while_loop(lambda b: b < nb, body, jnp.int32(0))
    gmax = gm_ref[...]
    gmax = jnp.where(jnp.isfinite(gmax), gmax, 0.0)

    ids = jax.lax.broadcasted_iota(jnp.int32, (nb, n), 1)
    oh_h = (ids == heads_ref[...]).astype(jnp.float32)
    oh_t = (ids == tails_ref[...]).astype(jnp.float32)
    bh = jnp.dot(oh_h, bv)
    bt = jnp.dot(oh_t, bv)
    g_G = jnp.concatenate([gmax, bh, bt], axis=1)
    hmid = jnp.maximum(jnp.dot(g_G, Wg1_ref[...]) + bg1_ref[...], 0.0)
    logits = jnp.dot(hmid, Wg2_ref[...]) + bg2_ref[...]
    w = jax.nn.softmax(logits, axis=0)
    gall_ref[...] = jnp.sum(w * g_G, axis=0, keepdims=True)


def _final(bv, batch, heads, tails, Wg1, bg1, Wg2, bg2):
    n = bv.shape[0]
    nb = heads.shape[0]
    return pl.pallas_call(
        _final_body,
        in_specs=[
            pl.BlockSpec((n, H), lambda: (0, 0)),
            pl.BlockSpec((n, 1), lambda: (0, 0)),
            pl.BlockSpec((nb, 1), lambda: (0, 0)),
            pl.BlockSpec((nb, 1), lambda: (0, 0)),
            pl.BlockSpec((3 * H, H), lambda: (0, 0)),
            pl.BlockSpec((1, H), lambda: (0, 0)),
            pl.BlockSpec((H, 1), lambda: (0, 0)),
            pl.BlockSpec((1, 1), lambda: (0, 0)),
        ],
        out_specs=pl.BlockSpec((1, 3 * H), lambda: (0, 0)),
        out_shape=jax.ShapeDtypeStruct((1, 3 * H), jnp.float32),
        scratch_shapes=[pltpu.VMEM((nb, H), jnp.float32)],
    )(bv, batch.reshape(n, 1), heads.reshape(nb, 1), tails.reshape(nb, 1),
      Wg1, bg1.reshape(1, H), Wg2, bg2.reshape(1, 1))


def kernel(edge_features, edge_index, node_degrees, batch, heads, tails,
           g_all_init, Wa1, ba1, Wa2, ba2, Wu1, bu1, Wu2, bu2,
           Wg1, bg1, Wg2, bg2, We, be):
    e = edge_features.shape[0]
    n = node_degrees.shape[0]
    f32 = jnp.float32
    with jax.enable_x64(False):
        (edge_features, node_degrees, g_all_init, Wa1, ba1, Wa2, ba2, Wu1,
         bu1, Wu2, bu2, Wg1, bg1, Wg2, bg2, We, be) = jax.tree.map(
            lambda x: x.astype(f32),
            (edge_features, node_degrees, g_all_init, Wa1, ba1, Wa2, ba2, Wu1,
             bu1, Wu2, bu2, Wg1, bg1, Wg2, bg2, We, be))
        src = edge_index[0].astype(jnp.int32)
        dst = edge_index[1].astype(jnp.int32)
        heads32 = heads.astype(jnp.int32)
        tails32 = tails.astype(jnp.int32)
        batch32 = batch.astype(jnp.int32)

        Wsrc, whsrc, wtsrc = Wu1[0:H], Wu1[H:H + 1], Wu1[H + 1:H + 2]
        Wdst, whdst, wtdst = (Wu1[H + 2:2 * H + 2], Wu1[2 * H + 2:2 * H + 3],
                              Wu1[2 * H + 3:2 * H + 4])
        Wue = Wu1[2 * H + 4:]

        b_e = jnp.concatenate(
            [edge_features,
             jnp.broadcast_to(g_all_init, (e, g_all_init.shape[-1]))], axis=1)

        bv = None
        for it in range(2):
            V, exv = _attn(b_e, Wa1, ba1, Wa2, ba2)
            S = jax.ops.segment_sum(V, dst, num_segments=n)
            s = jax.ops.segment_sum(exv, dst, num_segments=n)
            Q, R, bv = _node_tables(S, s, node_degrees, heads32, tails32,
                                    Wsrc, whsrc, wtsrc, Wdst, whdst, wtdst)
            g = _sc_gather_add(Q, R, src, dst)
            if it == 0:
                b_e = _update(g, b_e, Wue, bu1, Wu2, bu2)
            else:
                w_e = _update_we(g, b_e, Wue, bu1, Wu2, bu2, We, be)

        g_all = _final(bv, batch32, heads32, tails32, Wg1, bg1, Wg2, bg2)
    return (w_e.astype(jnp.float64), g_all.astype(jnp.float64))


# speedup vs baseline: 75.9208x; 1.3347x over previous
import functools

import jax
import jax.numpy as jnp
from jax import lax
from jax.experimental import pallas as pl
from jax.experimental.pallas import tpu as pltpu
from jax.experimental.pallas import tpu_sc as plsc

H = 128
BE = 2000
BN = 2000


_NC = 2
_NS = 16
_NW = _NC * _NS
_CK = 128


def _sc_gather_add(Q, R, src, dst):
    e = src.shape[0]
    per_w = e // _NW
    nchunk = per_w // _CK
    tail = per_w - nchunk * _CK
    mesh = plsc.VectorSubcoreMesh(core_axis_name="c", subcore_axis_name="s")

    @functools.partial(
        pl.kernel, mesh=mesh,
        out_type=jax.ShapeDtypeStruct((e, H), jnp.float32),
        scratch_types=[
            pltpu.VMEM((_CK,), jnp.int32),
            pltpu.VMEM((_CK,), jnp.int32),
            pltpu.VMEM((_CK, H), jnp.float32),
            pltpu.VMEM((_CK, H), jnp.float32),
            pltpu.SemaphoreType.DMA,
            pltpu.SemaphoreType.DMA,
        ],
    )
    def k(q_hbm, r_hbm, src_hbm, dst_hbm, out_hbm,
          idx_s, idx_d, rows_q, rows_d, sem1, sem2):
        wid = lax.axis_index("s") * _NC + lax.axis_index("c")
        base = wid * per_w

        def do_chunk(cb, ck):
            isl = idx_s.at[pl.ds(0, ck)]
            idl = idx_d.at[pl.ds(0, ck)]
            rq = rows_q.at[pl.ds(0, ck)]
            rd = rows_d.at[pl.ds(0, ck)]
            pltpu.sync_copy(src_hbm.at[pl.ds(cb, ck)], isl)
            pltpu.sync_copy(dst_hbm.at[pl.ds(cb, ck)], idl)
            cp1 = pltpu.async_copy(q_hbm.at[isl], rq, sem1)
            cp2 = pltpu.async_copy(r_hbm.at[idl], rd, sem2)
            cp1.wait()
            cp2.wait()

            def row_body(rr, c):
                for j in range(H // 16):
                    sl = pl.ds(j * 16, 16)
                    rows_q[rr, sl] = rows_q[rr, sl] + rows_d[rr, sl]
                return c

            lax.fori_loop(0, ck, row_body, 0)
            pltpu.sync_copy(rq, out_hbm.at[pl.ds(cb, ck)])

        def chunk_body(kk, c):
            do_chunk(base + kk * _CK, _CK)
            return c

        lax.fori_loop(0, nchunk, chunk_body, 0)
        if tail:
            do_chunk(base + nchunk * _CK, tail)

    return k(Q, R, src, dst)


_WV = 144


def _sc_scatter_add(V, dst, n):
    e = dst.shape[0]
    per_w = e // _NW
    nchunk = per_w // _CK
    tail = per_w - nchunk * _CK
    n_pad = ((n + 127) // 128) * 128
    rows_per_tile = n_pad // _NS
    mesh = plsc.VectorSubcoreMesh(core_axis_name="c", subcore_axis_name="s")

    @functools.partial(
        pl.kernel, mesh=mesh,
        compiler_params=pltpu.CompilerParams(use_tc_tiling_on_sc=False),
        out_type=jax.ShapeDtypeStruct((_NC, n_pad, _WV), jnp.float32),
        scratch_types=[
            pltpu.VMEM((_CK,), jnp.int32),
            pltpu.VMEM((8,), jnp.int32),
            pltpu.VMEM((_CK, _WV), jnp.float32),
            pltpu.VMEM((8, _WV), jnp.float32),
            pltpu.VMEM((128, _WV), jnp.float32),
            pltpu.VMEM_SHARED((n_pad, _WV), jnp.float32),
        ],
    )
    def k(v_hbm, dst_hbm, out_hbm, idx_v, idx_t, rows_v, rows_t, zbuf, acc):
        cc = lax.axis_index("c")
        ss = lax.axis_index("s")
        wid = ss * _NC + cc
        base = wid * per_w

        zv = jnp.zeros((16,), jnp.float32)

        def zrow(rr, c):
            for j in range(_WV // 16):
                zbuf[rr, pl.ds(j * 16, 16)] = zv
            return c

        lax.fori_loop(0, 128, zrow, 0)
        for p in range(0, rows_per_tile, 128):
            blk = min(128, rows_per_tile - p)
            pltpu.sync_copy(zbuf.at[pl.ds(0, blk)],
                            acc.at[pl.ds(ss * rows_per_tile + p, blk)])
        plsc.subcore_barrier()

        def chunk(cb, idx, rows):
            ck = rows.shape[0]
            pltpu.sync_copy(dst_hbm.at[pl.ds(cb, ck)], idx)
            pltpu.sync_copy(v_hbm.at[pl.ds(cb, ck)], rows)
            pltpu.sync_copy(rows, acc.at[idx], add=True)

        def chunk_body(kk, c):
            chunk(base + kk * _CK, idx_v, rows_v)
            return c

        lax.fori_loop(0, nchunk, chunk_body, 0)
        if tail:
            chunk(base + nchunk * _CK, idx_t, rows_t)
        plsc.subcore_barrier()

        pltpu.sync_copy(acc.at[pl.ds(ss * rows_per_tile, rows_per_tile)],
                        out_hbm.at[cc, pl.ds(ss * rows_per_tile, rows_per_tile)])

    return k(V, dst)


def _attn_body(be_ref, Wa1_ref, ba1_ref, Wa2_ref, ba2_ref, V_ref):
    x = be_ref[...]
    h = jnp.maximum(jnp.dot(x, Wa1_ref[...]) + ba1_ref[...], 0.0)
    a = jax.nn.sigmoid(jnp.dot(h, Wa2_ref[...]) + ba2_ref[...])
    ex = jnp.exp(a)
    V_ref[...] = jnp.concatenate(
        [x * ex, jnp.broadcast_to(ex, (x.shape[0], _WV - H))], axis=1)


def _attn(b_e, Wa1, ba1, Wa2, ba2):
    e = b_e.shape[0]
    grid = e // BE
    return pl.pallas_call(
        _attn_body,
        grid=(grid,),
        in_specs=[
            pl.BlockSpec((BE, H), lambda i: (i, 0)),
            pl.BlockSpec((H, H), lambda i: (0, 0)),
            pl.BlockSpec((1, H), lambda i: (0, 0)),
            pl.BlockSpec((H, 1), lambda i: (0, 0)),
            pl.BlockSpec((1, 1), lambda i: (0, 0)),
        ],
        out_specs=pl.BlockSpec((BE, _WV), lambda i: (i, 0)),
        out_shape=jax.ShapeDtypeStruct((e, _WV), jnp.float32),
    )(b_e, Wa1, ba1.reshape(1, H), Wa2, ba2.reshape(1, 1))


def _node_body(P0_ref, P1_ref, deg_ref, heads_ref, tails_ref,
               Wsrc_ref, whsrc_ref, wtsrc_ref, Wdst_ref, whdst_ref, wtdst_ref,
               Q_ref, R_ref, bv_ref):
    i = pl.program_id(0)
    P = P0_ref[...] + P1_ref[...]
    S = P[:, 0:H]
    s1 = P[:, H:H + 1]
    deg = deg_ref[...]
    bv = jnp.where(s1 > 0.0, S / jnp.where(s1 > 0.0, s1, 1.0), 0.0)
    bv = bv / (1.0 + deg)
    ids = i * BN + jax.lax.broadcasted_iota(jnp.int32, (BN, 1), 0)
    ih = jnp.minimum(jnp.sum((ids == heads_ref[...]).astype(jnp.float32),
                             axis=1, keepdims=True), 1.0)
    it = jnp.minimum(jnp.sum((ids == tails_ref[...]).astype(jnp.float32),
                             axis=1, keepdims=True), 1.0)
    Q_ref[...] = jnp.dot(bv, Wsrc_ref[...]) + ih * whsrc_ref[...] + it * wtsrc_ref[...]
    R_ref[...] = jnp.dot(bv, Wdst_ref[...]) + ih * whdst_ref[...] + it * wtdst_ref[...]
    bv_ref[...] = bv


def _node_tables(P0, P1, deg, heads, tails, Wsrc, whsrc, wtsrc, Wdst, whdst, wtdst):
    n = deg.shape[0]
    nb = heads.shape[0]
    grid = n // BN
    return pl.pallas_call(
        _node_body,
        grid=(grid,),
        in_specs=[
            pl.BlockSpec((BN, _WV), lambda i: (i, 0)),
            pl.BlockSpec((BN, _WV), lambda i: (i, 0)),
            pl.BlockSpec((BN, 1), lambda i: (i, 0)),
            pl.BlockSpec((1, nb), lambda i: (0, 0)),
            pl.BlockSpec((1, nb), lambda i: (0, 0)),
            pl.BlockSpec((H, H), lambda i: (0, 0)),
            pl.BlockSpec((1, H), lambda i: (0, 0)),
            pl.BlockSpec((1, H), lambda i: (0, 0)),
            pl.BlockSpec((H, H), lambda i: (0, 0)),
            pl.BlockSpec((1, H), lambda i: (0, 0)),
            pl.BlockSpec((1, H), lambda i: (0, 0)),
        ],
        out_specs=[
            pl.BlockSpec((BN, H), lambda i: (i, 0)),
            pl.BlockSpec((BN, H), lambda i: (i, 0)),
            pl.BlockSpec((BN, H), lambda i: (i, 0)),
        ],
        out_shape=[
            jax.ShapeDtypeStruct((n, H), jnp.float32),
            jax.ShapeDtypeStruct((n, H), jnp.float32),
            jax.ShapeDtypeStruct((n, H), jnp.float32),
        ],
    )(P0, P1, deg.reshape(n, 1), heads.reshape(1, nb), tails.reshape(1, nb),
      Wsrc, whsrc, wtsrc, Wdst, whdst, wtdst)


def _update_body(g_ref, be_ref, Wue_ref, bu1_ref, Wu2_ref, bu2_ref, out_ref):
    z = g_ref[...] + jnp.dot(be_ref[...], Wue_ref[...]) + bu1_ref[...]
    out_ref[...] = jnp.dot(jnp.maximum(z, 0.0), Wu2_ref[...]) + bu2_ref[...]


def _update(g, b_e, Wue, bu1, Wu2, bu2):
    e = g.shape[0]
    grid = e // BE
    return pl.pallas_call(
        _update_body,
        grid=(grid,),
        in_specs=[
            pl.BlockSpec((BE, H), lambda i: (i, 0)),
            pl.BlockSpec((BE, H), lambda i: (i, 0)),
            pl.BlockSpec((H, H), lambda i: (0, 0)),
            pl.BlockSpec((1, H), lambda i: (0, 0)),
            pl.BlockSpec((H, H), lambda i: (0, 0)),
            pl.BlockSpec((1, H), lambda i: (0, 0)),
        ],
        out_specs=pl.BlockSpec((BE, H), lambda i: (i, 0)),
        out_shape=jax.ShapeDtypeStruct((e, H), jnp.float32),
    )(g, b_e, Wue, bu1.reshape(1, H), Wu2, bu2.reshape(1, H))


def _update_we_body(g_ref, be_ref, Wue_ref, bu1_ref, Wu2_ref, bu2_ref,
                    We_ref, be2_ref, we_ref):
    z = g_ref[...] + jnp.dot(be_ref[...], Wue_ref[...]) + bu1_ref[...]
    bnew = jnp.dot(jnp.maximum(z, 0.0), Wu2_ref[...]) + bu2_ref[...]
    we_ref[...] = jax.nn.sigmoid(jnp.dot(bnew, We_ref[...]) + be2_ref[...])


def _update_we(g, b_e, Wue, bu1, Wu2, bu2, We, be):
    e = g.shape[0]
    grid = e // BE
    return pl.pallas_call(
        _update_we_body,
        grid=(grid,),
        in_specs=[
            pl.BlockSpec((BE, H), lambda i: (i, 0)),
            pl.BlockSpec((BE, H), lambda i: (i, 0)),
            pl.BlockSpec((H, H), lambda i: (0, 0)),
            pl.BlockSpec((1, H), lambda i: (0, 0)),
            pl.BlockSpec((H, H), lambda i: (0, 0)),
            pl.BlockSpec((1, H), lambda i: (0, 0)),
            pl.BlockSpec((H, 1), lambda i: (0, 0)),
            pl.BlockSpec((1, 1), lambda i: (0, 0)),
        ],
        out_specs=pl.BlockSpec((BE, 1), lambda i: (i, 0)),
        out_shape=jax.ShapeDtypeStruct((e, 1), jnp.float32),
    )(g, b_e, Wue, bu1.reshape(1, H), Wu2, bu2.reshape(1, H),
      We, be.reshape(1, 1))


def _final_body(bv_ref, batch_ref, heads_ref, tails_ref,
                Wg1_ref, bg1_ref, Wg2_ref, bg2_ref, gall_ref, gm_ref):
    bv = bv_ref[...]
    n = bv.shape[0]
    nb = heads_ref.shape[0]
    batch = batch_ref[...]

    def body(b):
        mask = batch == b
        m = jnp.max(jnp.where(mask, bv, -jnp.inf), axis=0)
        gm_ref[pl.ds(b, 1), :] = m[None, :]
        return b + jnp.int32(1)

    jax.lax.while_loop(lambda b: b < nb, body, jnp.int32(0))
    gmax = gm_ref[...]
    gmax = jnp.where(jnp.isfinite(gmax), gmax, 0.0)

    ids = jax.lax.broadcasted_iota(jnp.int32, (nb, n), 1)
    oh_h = (ids == heads_ref[...]).astype(jnp.float32)
    oh_t = (ids == tails_ref[...]).astype(jnp.float32)
    bh = jnp.dot(oh_h, bv)
    bt = jnp.dot(oh_t, bv)
    g_G = jnp.concatenate([gmax, bh, bt], axis=1)
    hmid = jnp.maximum(jnp.dot(g_G, Wg1_ref[...]) + bg1_ref[...], 0.0)
    logits = jnp.dot(hmid, Wg2_ref[...]) + bg2_ref[...]
    w = jax.nn.softmax(logits, axis=0)
    gall_ref[...] = jnp.sum(w * g_G, axis=0, keepdims=True)


def _final(bv, batch, heads, tails, Wg1, bg1, Wg2, bg2):
    n = bv.shape[0]
    nb = heads.shape[0]
    return pl.pallas_call(
        _final_body,
        in_specs=[
            pl.BlockSpec((n, H), lambda: (0, 0)),
            pl.BlockSpec((n, 1), lambda: (0, 0)),
            pl.BlockSpec((nb, 1), lambda: (0, 0)),
            pl.BlockSpec((nb, 1), lambda: (0, 0)),
            pl.BlockSpec((3 * H, H), lambda: (0, 0)),
            pl.BlockSpec((1, H), lambda: (0, 0)),
            pl.BlockSpec((H, 1), lambda: (0, 0)),
            pl.BlockSpec((1, 1), lambda: (0, 0)),
        ],
        out_specs=pl.BlockSpec((1, 3 * H), lambda: (0, 0)),
        out_shape=jax.ShapeDtypeStruct((1, 3 * H), jnp.float32),
        scratch_shapes=[pltpu.VMEM((nb, H), jnp.float32)],
    )(bv, batch.reshape(n, 1), heads.reshape(nb, 1), tails.reshape(nb, 1),
      Wg1, bg1.reshape(1, H), Wg2, bg2.reshape(1, 1))


def kernel(edge_features, edge_index, node_degrees, batch, heads, tails,
           g_all_init, Wa1, ba1, Wa2, ba2, Wu1, bu1, Wu2, bu2,
           Wg1, bg1, Wg2, bg2, We, be):
    e = edge_features.shape[0]
    n = node_degrees.shape[0]
    f32 = jnp.float32
    with jax.enable_x64(False):
        (edge_features, node_degrees, g_all_init, Wa1, ba1, Wa2, ba2, Wu1,
         bu1, Wu2, bu2, Wg1, bg1, Wg2, bg2, We, be) = jax.tree.map(
            lambda x: x.astype(f32),
            (edge_features, node_degrees, g_all_init, Wa1, ba1, Wa2, ba2, Wu1,
             bu1, Wu2, bu2, Wg1, bg1, Wg2, bg2, We, be))
        src = edge_index[0].astype(jnp.int32)
        dst = edge_index[1].astype(jnp.int32)
        heads32 = heads.astype(jnp.int32)
        tails32 = tails.astype(jnp.int32)
        batch32 = batch.astype(jnp.int32)

        Wsrc, whsrc, wtsrc = Wu1[0:H], Wu1[H:H + 1], Wu1[H + 1:H + 2]
        Wdst, whdst, wtdst = (Wu1[H + 2:2 * H + 2], Wu1[2 * H + 2:2 * H + 3],
                              Wu1[2 * H + 3:2 * H + 4])
        Wue = Wu1[2 * H + 4:]

        b_e = jnp.concatenate(
            [edge_features,
             jnp.broadcast_to(g_all_init, (e, g_all_init.shape[-1]))], axis=1)

        bv = None
        for it in range(2):
            V = _attn(b_e, Wa1, ba1, Wa2, ba2)
            P = _sc_scatter_add(V, dst, n)
            Q, R, bv = _node_tables(P[0], P[1], node_degrees, heads32, tails32,
                                    Wsrc, whsrc, wtsrc, Wdst, whdst, wtdst)
            g = _sc_gather_add(Q, R, src, dst)
            if it == 0:
                b_e = _update(g, b_e, Wue, bu1, Wu2, bu2)
            else:
                w_e = _update_we(g, b_e, Wue, bu1, Wu2, bu2, We, be)

        g_all = _final(bv, batch32, heads32, tails32, Wg1, bg1, Wg2, bg2)
    return (w_e.astype(jnp.float64), g_all.astype(jnp.float64))
